# baseline (speedup 1.0000x reference)
import functools

import jax
import jax.numpy as jnp
from jax import lax
from jax.experimental import pallas as pl
from jax.experimental.pallas import tpu as pltpu
from jax.experimental.pallas import tpu_sc as plsc

N = 10000
E = 160000
H = 100
HP = 128
DE = 16
C = 128
PE = 163840
NP = 10240
NC, NS, NW = 2, 16, 32
TE = 1024


def _pad2(w, r, c):
  return jnp.pad(w, ((0, r - w.shape[0]), (0, c - w.shape[1])))


def _pad1(b, c):
  return jnp.pad(b, (0, c - b.shape[0])).reshape(1, c)


def _sc_mesh():
  return plsc.VectorSubcoreMesh(
      core_axis_name="c", subcore_axis_name="s", num_cores=NC,
      num_subcores=NS)


NBUF = 2


def _sc_gather(table, idx2, rows):
  rpw = rows // NW
  stripe = NP // NS

  def body(tab_h, idx_h, out_h, tab_sh, idx_v, rows_v, gsem, wsem):
    c = lax.axis_index("c")
    s = lax.axis_index("s")
    wid = c * NS + s
    pltpu.sync_copy(tab_h.at[pl.ds(s * stripe, stripe)],
                    tab_sh.at[pl.ds(s * stripe, stripe)])
    pltpu.sync_copy(idx_h.at[pl.ds(wid * rpw, rpw)], idx_v)
    plsc.subcore_barrier()
    for b in range(NBUF):
      pltpu.async_copy(tab_sh.at[idx_v.at[b]], rows_v.at[b], gsem.at[b])

    def step(j, carry):
      slot = lax.rem(j, NBUF)
      dst = out_h.at[pl.ds((wid * rpw + j) * C, C)]
      pltpu.make_async_copy(tab_sh.at[idx_v.at[j]], rows_v.at[slot],
                            gsem.at[slot]).wait()
      pltpu.async_copy(rows_v.at[slot], dst, wsem.at[slot])
      k = j + NBUF

      @pl.when(k < rpw)
      def _():
        pltpu.make_async_copy(rows_v.at[slot], dst, wsem.at[slot]).wait()
        pltpu.async_copy(tab_sh.at[idx_v.at[k]], rows_v.at[slot],
                         gsem.at[slot])

      return carry

    lax.fori_loop(0, rpw, step, 0)
    for b in range(NBUF):
      pltpu.make_async_copy(rows_v.at[b], out_h.at[pl.ds(0, C)],
                            wsem.at[b]).wait()

  f = pl.kernel(
      body,
      out_type=jax.ShapeDtypeStruct((rows * C, HP), jnp.float32),
      mesh=_sc_mesh(),
      scratch_types=[
          pltpu.VMEM_SHARED((NP, HP), jnp.float32),
          pltpu.VMEM((rpw, C), jnp.int32),
          pltpu.VMEM((NBUF, C, HP), jnp.float32),
          pltpu.SemaphoreType.DMA((NBUF,)),
          pltpu.SemaphoreType.DMA((NBUF,)),
      ],
  )
  return f(table, idx2)


def _sc_scatter_add(m, dst2, zeros):
  rows = PE // C
  rpw = rows // NW
  stripe = NP // NS

  def body(m_h, dst_h, z_h, out_h, acc_sh, idx_v, buf_v, rsem):
    c = lax.axis_index("c")
    s = lax.axis_index("s")
    wid = c * NS + s
    pltpu.sync_copy(z_h.at[pl.ds(s * stripe, stripe)],
                    acc_sh.at[pl.ds(s * stripe, stripe)])
    pltpu.sync_copy(dst_h.at[pl.ds(wid * rpw, rpw)], idx_v)
    plsc.subcore_barrier()
    pltpu.async_copy(m_h.at[pl.ds(wid * rpw * C, C)], buf_v.at[0],
                     rsem.at[0])

    def step(j, carry):
      slot = lax.rem(j, 2)
      row0 = (wid * rpw + j) * C
      pltpu.make_async_copy(m_h.at[pl.ds(row0, C)], buf_v.at[slot],
                            rsem.at[slot]).wait()
      k = j + 1

      @pl.when(k < rpw)
      def _():
        pltpu.async_copy(m_h.at[pl.ds((wid * rpw + k) * C, C)],
                         buf_v.at[1 - slot], rsem.at[1 - slot])

      pltpu.sync_copy(buf_v.at[slot], acc_sh.at[idx_v.at[j]], add=True)
      return carry

    lax.fori_loop(0, rpw, step, 0)
    plsc.subcore_barrier()
    pltpu.sync_copy(acc_sh.at[pl.ds(s * stripe, stripe)],
                    out_h.at[c, pl.ds(s * stripe, stripe)])

  f = pl.kernel(
      body,
      out_type=jax.ShapeDtypeStruct((NC, NP, HP), jnp.float32),
      mesh=_sc_mesh(),
      scratch_types=[
          pltpu.VMEM_SHARED((NP, HP), jnp.float32),
          pltpu.VMEM((rpw, C), jnp.int32),
          pltpu.VMEM((2, C, HP), jnp.float32),
          pltpu.SemaphoreType.DMA((2,)),
      ],
  )
  return f(m, dst2, zeros)


def _full(shape):
  return pl.BlockSpec(shape, lambda *i: (0,) * len(shape))


def _bdot(a, b):
  return jnp.dot(a.astype(jnp.bfloat16), b.astype(jnp.bfloat16),
                 preferred_element_type=jnp.float32)


def _node_embed_k(x_ref, w_ref, b_ref, o_ref):
  h = jnp.dot(x_ref[...], w_ref[...],
              preferred_element_type=jnp.float32) + b_ref[...]
  o_ref[:N, :] = h
  o_ref[N:, :] = jnp.zeros((NP - N, HP), jnp.float32)


def _node_embed(x, w, b):
  return pl.pallas_call(
      _node_embed_k,
      out_shape=jax.ShapeDtypeStruct((NP, HP), jnp.float32),
      in_specs=[_full((N, HP)), _full((HP, HP)), _full((1, HP))],
      out_specs=_full((NP, HP)),
  )(x, w, b)


def _edge_embed_msg_k(ea_ref, hs_ref, ew_ref, eb_ref, lw_ref, lb_ref,
                      e_ref, m_ref):
  e = jnp.dot(ea_ref[...], ew_ref[...],
              preferred_element_type=jnp.float32) + eb_ref[...]
  e_ref[...] = e
  m_ref[...] = jnp.maximum(
      hs_ref[...].astype(jnp.float32) + _bdot(e, lw_ref[...]) + lb_ref[...],
      0.0)


def _edge_embed_msg(ea, hs, ew, eb, lw, lb):
  g = PE // TE
  row = pl.BlockSpec((TE, HP), lambda i: (i, 0))
  return pl.pallas_call(
      _edge_embed_msg_k,
      grid=(g,),
      out_shape=[jax.ShapeDtypeStruct((PE, HP), jnp.float32),
                 jax.ShapeDtypeStruct((PE, HP), jnp.float32)],
      in_specs=[pl.BlockSpec((TE, DE), lambda i: (i, 0)), row,
                _full((DE, HP)), _full((1, HP)), _full((HP, HP)),
                _full((1, HP))],
      out_specs=[row, row],
  )(ea, hs, ew, eb, lw, lb)


def _f32(x):
  return x.astype(jnp.float32)


def _node_update_k(h_ref, ag_ref, w1_ref, b1_ref, w2_ref, b2_ref,
                   g_ref, bb_ref, o_ref):
  h = h_ref[:N, :]
  z = h + ag_ref[0, :N, :] + ag_ref[1, :N, :]
  z = jnp.maximum(jnp.dot(z, w1_ref[...],
                          preferred_element_type=jnp.float32) + b1_ref[...],
                  0.0)
  z = jnp.dot(z, w2_ref[...],
              preferred_element_type=jnp.float32) + b2_ref[...]
  mu = jnp.mean(z, axis=0, keepdims=True)
  zc = z - mu
  var = jnp.mean(zc * zc, axis=0, keepdims=True)
  zn = zc * lax.rsqrt(var + 1e-5) * g_ref[...] + bb_ref[...]
  hn = (h + jnp.maximum(zn, 0.0)) * 0.5
  o_ref[:N, :] = hn
  o_ref[N:, :] = jnp.zeros((NP - N, HP), jnp.float32)


def _node_update(h, aggr, w1, b1, w2, b2, g, bb):
  return pl.pallas_call(
      _node_update_k,
      out_shape=jax.ShapeDtypeStruct((NP, HP), jnp.float32),
      in_specs=[_full((NP, HP)), _full((NC, NP, HP)), _full((HP, HP)),
                _full((1, HP)), _full((HP, HP)), _full((1, HP)),
                _full((1, HP)), _full((1, HP))],
      out_specs=_full((NP, HP)),
  )(h, aggr, w1, b1, w2, b2, g, bb)


def _edge_update_msg_k(hs_ref, hd_ref, e_ref, w1a_ref, w1b_ref, w1c_ref,
                       b1_ref, w2_ref, b2_ref, lw_ref, lb_ref,
                       en_ref, m_ref):
  hs = hs_ref[...]
  e = e_ref[...]
  t = jnp.maximum(_bdot(hs, w1a_ref[...]) + _bdot(hd_ref[...], w1b_ref[...]) +
                  _bdot(e, w1c_ref[...]) + b1_ref[...], 0.0)
  en = e + (_bdot(t, w2_ref[...]) + b2_ref[...]) * 0.5
  en_ref[...] = en
  m_ref[...] = jnp.maximum(_f32(hs) + _bdot(en, lw_ref[...]) + lb_ref[...],
                           0.0)


def _edge_update_msg(hs, hd, e, w1a, w1b, w1c, b1, w2, b2, lw, lb):
  g = PE // TE
  row = pl.BlockSpec((TE, HP), lambda i: (i, 0))
  wspec = _full((HP, HP))
  bspec = _full((1, HP))
  return pl.pallas_call(
      _edge_update_msg_k,
      grid=(g,),
      out_shape=[jax.ShapeDtypeStruct((PE, HP), jnp.float32),
                 jax.ShapeDtypeStruct((PE, HP), jnp.float32)],
      in_specs=[row, row, row, wspec, wspec, wspec, bspec, wspec, bspec,
                wspec, bspec],
      out_specs=[row, row],
  )(hs, hd, e, w1a, w1b, w1c, b1, w2, b2, lw, lb)


def _final_k(hs_ref, hd_ref, e_ref, w1a_ref, w1b_ref, w1c_ref, b1_ref,
             w2_ref, b2_ref, m1a_ref, m1b_ref, m1c_ref, mb1_ref,
             mw2_ref, mb2_ref, mw3_ref, mb3_ref, o_ref):
  hs = hs_ref[...]
  hd = hd_ref[...]
  e = e_ref[...]
  t = jnp.maximum(_bdot(hs, w1a_ref[...]) + _bdot(hd, w1b_ref[...]) +
                  _bdot(e, w1c_ref[...]) + b1_ref[...], 0.0)
  e2 = e + (_bdot(t, w2_ref[...]) + b2_ref[...]) * 0.5
  o1 = jnp.maximum(_bdot(hs, m1a_ref[...]) + _bdot(hd, m1b_ref[...]) +
                   _bdot(e2, m1c_ref[...]) + mb1_ref[...], 0.0)
  o2 = jnp.maximum(_bdot(o1, mw2_ref[...]) + mb2_ref[...], 0.0)
  o3 = _bdot(o2, mw3_ref[...]) + mb3_ref[...]
  o_ref[...] = o3[:, :8]


def _final(hs, hd, e, w1a, w1b, w1c, b1, w2, b2, m1a, m1b, m1c, mb1,
           mw2, mb2, mw3, mb3):
  g = PE // TE
  row = pl.BlockSpec((TE, HP), lambda i: (i, 0))
  wspec = _full((HP, HP))
  bspec = _full((1, HP))
  return pl.pallas_call(
      _final_k,
      grid=(g,),
      out_shape=jax.ShapeDtypeStruct((PE, 8), jnp.float32),
      in_specs=[row, row, row, wspec, wspec, wspec, bspec, wspec, bspec,
                wspec, wspec, wspec, bspec, wspec, bspec, wspec, bspec],
      out_specs=pl.BlockSpec((TE, 8), lambda i: (i, 0)),
  )(hs, hd, e, w1a, w1b, w1c, b1, w2, b2, m1a, m1b, m1c, mb1, mw2, mb2,
    mw3, mb3)


def kernel(x, edge_index, edge_attr, node_w, node_b, edge_w, edge_b,
           conv_w1, conv_b1, conv_w2, conv_b2, lin_w, lin_b,
           emlp_w1, emlp_b1, emlp_w2, emlp_b2, bn_g, bn_b,
           mlp_w1, mlp_b1, mlp_w2, mlp_b2, mlp_w3, mlp_b3):
  src = edge_index[0].astype(jnp.int32)
  dst = edge_index[1].astype(jnp.int32)
  src2 = jnp.pad(src, (0, PE - E)).reshape(PE // C, C)
  dst2 = jnp.pad(dst, (0, PE - E),
                 constant_values=N).reshape(PE // C, C)
  sd2 = jnp.concatenate([src2, dst2], axis=0)
  ea = jnp.pad(edge_attr, ((0, PE - E), (0, 0)))
  zeros_np = jnp.zeros((NP, HP), jnp.float32)

  nw = _pad2(node_w, HP, HP)
  nb = _pad1(node_b, HP)
  ew = _pad2(edge_w, DE, HP)
  eb = _pad1(edge_b, HP)
  lw = [_pad2(lin_w[i], HP, HP) for i in range(2)]
  lb = [_pad1(lin_b[i], HP) for i in range(2)]
  cw1 = [_pad2(conv_w1[i], HP, HP) for i in range(2)]
  cb1 = [_pad1(conv_b1[i], HP) for i in range(2)]
  cw2 = [_pad2(conv_w2[i], HP, HP) for i in range(2)]
  cb2 = [_pad1(conv_b2[i], HP) for i in range(2)]
  g_ = [_pad1(bn_g[i], HP) for i in range(2)]
  bb = [_pad1(bn_b[i], HP) for i in range(2)]
  e1a = [_pad2(emlp_w1[i][:H], HP, HP) for i in range(2)]
  e1b = [_pad2(emlp_w1[i][H:2 * H], HP, HP) for i in range(2)]
  e1c = [_pad2(emlp_w1[i][2 * H:], HP, HP) for i in range(2)]
  eb1 = [_pad1(emlp_b1[i], HP) for i in range(2)]
  ew2 = [_pad2(emlp_w2[i], HP, HP) for i in range(2)]
  eb2 = [_pad1(emlp_b2[i], HP) for i in range(2)]
  m1a = _pad2(mlp_w1[:H], HP, HP)
  m1b = _pad2(mlp_w1[H:2 * H], HP, HP)
  m1c = _pad2(mlp_w1[2 * H:], HP, HP)
  mb1 = _pad1(mlp_b1, HP)
  mw2 = _pad2(mlp_w2, HP, HP)
  mb2 = _pad1(mlp_b2, HP)
  mw3 = _pad2(mlp_w3, HP, HP)
  mb3 = _pad1(mlp_b3, HP)

  h = _node_embed(x, nw, nb)
  hs = _sc_gather(h, src2, PE // C)
  e, m = _edge_embed_msg(ea, hs, ew, eb, lw[0], lb[0])
  aggr = _sc_scatter_add(m, dst2, zeros_np)
  h = _node_update(h, aggr, cw1[0], cb1[0], cw2[0], cb2[0], g_[0], bb[0])

  hsd = _sc_gather(h, sd2, 2 * (PE // C))
  hs, hd = hsd[:PE], hsd[PE:]
  e, m = _edge_update_msg(hs, hd, e, e1a[0], e1b[0], e1c[0], eb1[0],
                          ew2[0], eb2[0], lw[1], lb[1])
  aggr = _sc_scatter_add(m, dst2, zeros_np)
  h = _node_update(h, aggr, cw1[1], cb1[1], cw2[1], cb2[1], g_[1], bb[1])

  hsd = _sc_gather(h, sd2, 2 * (PE // C))
  hs, hd = hsd[:PE], hsd[PE:]
  out = _final(hs, hd, e, e1a[1], e1b[1], e1c[1], eb1[1], ew2[1], eb2[1],
               m1a, m1b, m1c, mb1, mw2, mb2, mw3, mb3)
  return out[:E, 0]


# speedup vs baseline: 1.2260x; 1.2260x over previous
import functools

import jax
import jax.numpy as jnp
from jax import lax
from jax.experimental import pallas as pl
from jax.experimental.pallas import tpu as pltpu
from jax.experimental.pallas import tpu_sc as plsc

N = 10000
E = 160000
H = 100
HP = 128
DE = 16
C = 128
PE = 163840
NP = 10240
NC, NS, NW = 2, 16, 32
TE = 1024


def _pad2(w, r, c):
  return jnp.pad(w, ((0, r - w.shape[0]), (0, c - w.shape[1])))


def _pad1(b, c):
  return jnp.pad(b, (0, c - b.shape[0])).reshape(1, c)


def _sc_mesh():
  return plsc.VectorSubcoreMesh(
      core_axis_name="c", subcore_axis_name="s", num_cores=NC,
      num_subcores=NS)


NBUF = 2


def _sc_gather(table, idx2, nout):
  rows = nout * (PE // C)
  rpw = rows // NW
  stripe = NP // NS

  def body(tab_h, idx_h, *rest):
    outs = rest[:nout]
    tab_sh, idx_v, rows_v, gsem, wsem = rest[nout:]
    c = lax.axis_index("c")
    s = lax.axis_index("s")
    wid = c * NS + s
    pltpu.sync_copy(tab_h.at[pl.ds(s * stripe, stripe)],
                    tab_sh.at[pl.ds(s * stripe, stripe)])
    pltpu.sync_copy(idx_h.at[pl.ds(wid * rpw, rpw)], idx_v)
    plsc.subcore_barrier()
    for b in range(NBUF):
      pltpu.async_copy(tab_sh.at[idx_v.at[b]], rows_v.at[b], gsem.at[b])

    def step(j, carry):
      slot = lax.rem(j, NBUF)
      pltpu.make_async_copy(tab_sh.at[idx_v.at[j]], rows_v.at[slot],
                            gsem.at[slot]).wait()
      if nout == 1:
        pltpu.async_copy(rows_v.at[slot],
                         outs[0].at[pl.ds((wid * rpw + j) * C, C)],
                         wsem.at[slot])
      else:
        @pl.when(c == 0)
        def _():
          pltpu.async_copy(rows_v.at[slot],
                           outs[0].at[pl.ds((s * rpw + j) * C, C)],
                           wsem.at[slot])

        @pl.when(c == 1)
        def _():
          pltpu.async_copy(rows_v.at[slot],
                           outs[1].at[pl.ds((s * rpw + j) * C, C)],
                           wsem.at[slot])

      k = j + NBUF

      @pl.when(k < rpw)
      def _():
        pltpu.make_async_copy(rows_v.at[slot], outs[0].at[pl.ds(0, C)],
                              wsem.at[slot]).wait()
        pltpu.async_copy(tab_sh.at[idx_v.at[k]], rows_v.at[slot],
                         gsem.at[slot])

      return carry

    lax.fori_loop(0, rpw, step, 0)
    for b in range(NBUF):
      pltpu.make_async_copy(rows_v.at[b], outs[0].at[pl.ds(0, C)],
                            wsem.at[b]).wait()

  f = pl.kernel(
      body,
      out_type=[jax.ShapeDtypeStruct((PE, HP), jnp.float32)] * nout,
      mesh=_sc_mesh(),
      scratch_types=[
          pltpu.VMEM_SHARED((NP, HP), jnp.float32),
          pltpu.VMEM((rpw, C), jnp.int32),
          pltpu.VMEM((NBUF, C, HP), jnp.float32),
          pltpu.SemaphoreType.DMA((NBUF,)),
          pltpu.SemaphoreType.DMA((NBUF,)),
      ],
  )
  return f(table, idx2)


def _sc_scatter_add(m, dst2, zeros):
  rows = PE // C
  rpw = rows // NW
  stripe = NP // NS

  def body(m_h, dst_h, z_h, out_h, acc_sh, idx_v, buf_v, rsem):
    c = lax.axis_index("c")
    s = lax.axis_index("s")
    wid = c * NS + s
    pltpu.sync_copy(z_h.at[pl.ds(s * stripe, stripe)],
                    acc_sh.at[pl.ds(s * stripe, stripe)])
    pltpu.sync_copy(dst_h.at[pl.ds(wid * rpw, rpw)], idx_v)
    plsc.subcore_barrier()
    pltpu.async_copy(m_h.at[pl.ds(wid * rpw * C, C)], buf_v.at[0],
                     rsem.at[0])

    def step(j, carry):
      slot = lax.rem(j, 2)
      row0 = (wid * rpw + j) * C
      pltpu.make_async_copy(m_h.at[pl.ds(row0, C)], buf_v.at[slot],
                            rsem.at[slot]).wait()
      k = j + 1

      @pl.when(k < rpw)
      def _():
        pltpu.async_copy(m_h.at[pl.ds((wid * rpw + k) * C, C)],
                         buf_v.at[1 - slot], rsem.at[1 - slot])

      pltpu.sync_copy(buf_v.at[slot], acc_sh.at[idx_v.at[j]], add=True)
      return carry

    lax.fori_loop(0, rpw, step, 0)
    plsc.subcore_barrier()
    pltpu.sync_copy(acc_sh.at[pl.ds(s * stripe, stripe)],
                    out_h.at[c, pl.ds(s * stripe, stripe)])

  f = pl.kernel(
      body,
      out_type=jax.ShapeDtypeStruct((NC, NP, HP), jnp.float32),
      mesh=_sc_mesh(),
      scratch_types=[
          pltpu.VMEM_SHARED((NP, HP), jnp.float32),
          pltpu.VMEM((rpw, C), jnp.int32),
          pltpu.VMEM((2, C, HP), jnp.float32),
          pltpu.SemaphoreType.DMA((2,)),
      ],
  )
  return f(m, dst2, zeros)


def _full(shape):
  return pl.BlockSpec(shape, lambda *i: (0,) * len(shape))


def _bdot(a, b):
  return jnp.dot(a.astype(jnp.bfloat16), b.astype(jnp.bfloat16),
                 preferred_element_type=jnp.float32)


def _node_embed_k(x_ref, w_ref, b_ref, o_ref):
  h = jnp.dot(x_ref[...], w_ref[...],
              preferred_element_type=jnp.float32) + b_ref[...]
  o_ref[:N, :] = h
  o_ref[N:, :] = jnp.zeros((NP - N, HP), jnp.float32)


def _node_embed(x, w, b):
  return pl.pallas_call(
      _node_embed_k,
      out_shape=jax.ShapeDtypeStruct((NP, HP), jnp.float32),
      in_specs=[_full((N, HP)), _full((HP, HP)), _full((1, HP))],
      out_specs=_full((NP, HP)),
  )(x, w, b)


def _edge_embed_msg_k(ea_ref, hs_ref, ew_ref, eb_ref, lw_ref, lb_ref,
                      e_ref, m_ref):
  e = jnp.dot(ea_ref[...], ew_ref[...],
              preferred_element_type=jnp.float32) + eb_ref[...]
  e_ref[...] = e
  m_ref[...] = jnp.maximum(
      hs_ref[...].astype(jnp.float32) + _bdot(e, lw_ref[...]) + lb_ref[...],
      0.0)


def _edge_embed_msg(ea, hs, ew, eb, lw, lb):
  g = PE // TE
  row = pl.BlockSpec((TE, HP), lambda i: (i, 0))
  return pl.pallas_call(
      _edge_embed_msg_k,
      grid=(g,),
      out_shape=[jax.ShapeDtypeStruct((PE, HP), jnp.float32),
                 jax.ShapeDtypeStruct((PE, HP), jnp.float32)],
      in_specs=[pl.BlockSpec((TE, DE), lambda i: (i, 0)), row,
                _full((DE, HP)), _full((1, HP)), _full((HP, HP)),
                _full((1, HP))],
      out_specs=[row, row],
  )(ea, hs, ew, eb, lw, lb)


def _f32(x):
  return x.astype(jnp.float32)


def _node_update_k(h_ref, ag_ref, w1_ref, b1_ref, w2_ref, b2_ref,
                   g_ref, bb_ref, o_ref):
  h = h_ref[:N, :]
  z = h + ag_ref[0, :N, :] + ag_ref[1, :N, :]
  z = jnp.maximum(jnp.dot(z, w1_ref[...],
                          preferred_element_type=jnp.float32) + b1_ref[...],
                  0.0)
  z = jnp.dot(z, w2_ref[...],
              preferred_element_type=jnp.float32) + b2_ref[...]
  mu = jnp.mean(z, axis=0, keepdims=True)
  zc = z - mu
  var = jnp.mean(zc * zc, axis=0, keepdims=True)
  zn = zc * lax.rsqrt(var + 1e-5) * g_ref[...] + bb_ref[...]
  hn = (h + jnp.maximum(zn, 0.0)) * 0.5
  o_ref[:N, :] = hn
  o_ref[N:, :] = jnp.zeros((NP - N, HP), jnp.float32)


def _node_update(h, aggr, w1, b1, w2, b2, g, bb):
  return pl.pallas_call(
      _node_update_k,
      out_shape=jax.ShapeDtypeStruct((NP, HP), jnp.float32),
      in_specs=[_full((NP, HP)), _full((NC, NP, HP)), _full((HP, HP)),
                _full((1, HP)), _full((HP, HP)), _full((1, HP)),
                _full((1, HP)), _full((1, HP))],
      out_specs=_full((NP, HP)),
  )(h, aggr, w1, b1, w2, b2, g, bb)


def _edge_update_msg_k(hs_ref, hd_ref, e_ref, w1a_ref, w1b_ref, w1c_ref,
                       b1_ref, w2_ref, b2_ref, lw_ref, lb_ref,
                       en_ref, m_ref):
  hs = hs_ref[...]
  e = e_ref[...]
  t = jnp.maximum(_bdot(hs, w1a_ref[...]) + _bdot(hd_ref[...], w1b_ref[...]) +
                  _bdot(e, w1c_ref[...]) + b1_ref[...], 0.0)
  en = e + (_bdot(t, w2_ref[...]) + b2_ref[...]) * 0.5
  en_ref[...] = en
  m_ref[...] = jnp.maximum(_f32(hs) + _bdot(en, lw_ref[...]) + lb_ref[...],
                           0.0)


def _edge_update_msg(hs, hd, e, w1a, w1b, w1c, b1, w2, b2, lw, lb):
  g = PE // TE
  row = pl.BlockSpec((TE, HP), lambda i: (i, 0))
  wspec = _full((HP, HP))
  bspec = _full((1, HP))
  return pl.pallas_call(
      _edge_update_msg_k,
      grid=(g,),
      out_shape=[jax.ShapeDtypeStruct((PE, HP), jnp.float32),
                 jax.ShapeDtypeStruct((PE, HP), jnp.float32)],
      in_specs=[row, row, row, wspec, wspec, wspec, bspec, wspec, bspec,
                wspec, bspec],
      out_specs=[row, row],
  )(hs, hd, e, w1a, w1b, w1c, b1, w2, b2, lw, lb)


def _final_k(hs_ref, hd_ref, e_ref, w1a_ref, w1b_ref, w1c_ref, b1_ref,
             w2_ref, b2_ref, m1a_ref, m1b_ref, m1c_ref, mb1_ref,
             mw2_ref, mb2_ref, mw3_ref, mb3_ref, o_ref):
  hs = hs_ref[...]
  hd = hd_ref[...]
  e = e_ref[...]
  t = jnp.maximum(_bdot(hs, w1a_ref[...]) + _bdot(hd, w1b_ref[...]) +
                  _bdot(e, w1c_ref[...]) + b1_ref[...], 0.0)
  e2 = e + (_bdot(t, w2_ref[...]) + b2_ref[...]) * 0.5
  o1 = jnp.maximum(_bdot(hs, m1a_ref[...]) + _bdot(hd, m1b_ref[...]) +
                   _bdot(e2, m1c_ref[...]) + mb1_ref[...], 0.0)
  o2 = jnp.maximum(_bdot(o1, mw2_ref[...]) + mb2_ref[...], 0.0)
  o3 = _bdot(o2, mw3_ref[...]) + mb3_ref[...]
  o_ref[...] = o3[:, 0].reshape(TE // HP, HP)


def _final(hs, hd, e, w1a, w1b, w1c, b1, w2, b2, m1a, m1b, m1c, mb1,
           mw2, mb2, mw3, mb3):
  g = PE // TE
  row = pl.BlockSpec((TE, HP), lambda i: (i, 0))
  wspec = _full((HP, HP))
  bspec = _full((1, HP))
  return pl.pallas_call(
      _final_k,
      grid=(g,),
      out_shape=jax.ShapeDtypeStruct((PE // HP, HP), jnp.float32),
      in_specs=[row, row, row, wspec, wspec, wspec, bspec, wspec, bspec,
                wspec, wspec, wspec, bspec, wspec, bspec, wspec, bspec],
      out_specs=pl.BlockSpec((TE // HP, HP), lambda i: (i, 0)),
  )(hs, hd, e, w1a, w1b, w1c, b1, w2, b2, m1a, m1b, m1c, mb1, mw2, mb2,
    mw3, mb3)


def kernel(x, edge_index, edge_attr, node_w, node_b, edge_w, edge_b,
           conv_w1, conv_b1, conv_w2, conv_b2, lin_w, lin_b,
           emlp_w1, emlp_b1, emlp_w2, emlp_b2, bn_g, bn_b,
           mlp_w1, mlp_b1, mlp_w2, mlp_b2, mlp_w3, mlp_b3):
  src = edge_index[0].astype(jnp.int32)
  dst = edge_index[1].astype(jnp.int32)
  src2 = jnp.pad(src, (0, PE - E)).reshape(PE // C, C)
  dst2 = jnp.pad(dst, (0, PE - E),
                 constant_values=N).reshape(PE // C, C)
  sd2 = jnp.concatenate([src2, dst2], axis=0)
  ea = jnp.pad(edge_attr, ((0, PE - E), (0, 0)))
  zeros_np = jnp.zeros((NP, HP), jnp.float32)

  nw = _pad2(node_w, HP, HP)
  nb = _pad1(node_b, HP)
  ew = _pad2(edge_w, DE, HP)
  eb = _pad1(edge_b, HP)
  lw = [_pad2(lin_w[i], HP, HP) for i in range(2)]
  lb = [_pad1(lin_b[i], HP) for i in range(2)]
  cw1 = [_pad2(conv_w1[i], HP, HP) for i in range(2)]
  cb1 = [_pad1(conv_b1[i], HP) for i in range(2)]
  cw2 = [_pad2(conv_w2[i], HP, HP) for i in range(2)]
  cb2 = [_pad1(conv_b2[i], HP) for i in range(2)]
  g_ = [_pad1(bn_g[i], HP) for i in range(2)]
  bb = [_pad1(bn_b[i], HP) for i in range(2)]
  e1a = [_pad2(emlp_w1[i][:H], HP, HP) for i in range(2)]
  e1b = [_pad2(emlp_w1[i][H:2 * H], HP, HP) for i in range(2)]
  e1c = [_pad2(emlp_w1[i][2 * H:], HP, HP) for i in range(2)]
  eb1 = [_pad1(emlp_b1[i], HP) for i in range(2)]
  ew2 = [_pad2(emlp_w2[i], HP, HP) for i in range(2)]
  eb2 = [_pad1(emlp_b2[i], HP) for i in range(2)]
  m1a = _pad2(mlp_w1[:H], HP, HP)
  m1b = _pad2(mlp_w1[H:2 * H], HP, HP)
  m1c = _pad2(mlp_w1[2 * H:], HP, HP)
  mb1 = _pad1(mlp_b1, HP)
  mw2 = _pad2(mlp_w2, HP, HP)
  mb2 = _pad1(mlp_b2, HP)
  mw3 = _pad2(mlp_w3, HP, HP)
  mb3 = _pad1(mlp_b3, HP)

  h = _node_embed(x, nw, nb)
  hs, = _sc_gather(h, src2, 1)
  e, m = _edge_embed_msg(ea, hs, ew, eb, lw[0], lb[0])
  aggr = _sc_scatter_add(m, dst2, zeros_np)
  h = _node_update(h, aggr, cw1[0], cb1[0], cw2[0], cb2[0], g_[0], bb[0])

  hs, hd = _sc_gather(h, sd2, 2)
  e, m = _edge_update_msg(hs, hd, e, e1a[0], e1b[0], e1c[0], eb1[0],
                          ew2[0], eb2[0], lw[1], lb[1])
  aggr = _sc_scatter_add(m, dst2, zeros_np)
  h = _node_update(h, aggr, cw1[1], cb1[1], cw2[1], cb2[1], g_[1], bb[1])

  hs, hd = _sc_gather(h, sd2, 2)
  out = _final(hs, hd, e, e1a[1], e1b[1], e1c[1], eb1[1], ew2[1], eb2[1],
               m1a, m1b, m1c, mb1, mw2, mb2, mw3, mb3)
  return out.reshape(PE)[:E]


# speedup vs baseline: 1.2433x; 1.0141x over previous
import functools

import jax
import jax.numpy as jnp
from jax import lax
from jax.experimental import pallas as pl
from jax.experimental.pallas import tpu as pltpu
from jax.experimental.pallas import tpu_sc as plsc

N = 10000
E = 160000
H = 100
HP = 128
DE = 16
C = 128
PE = 163840
NP = 10240
NC, NS, NW = 2, 16, 32
TE = 1024


def _pad2(w, r, c):
  return jnp.pad(w, ((0, r - w.shape[0]), (0, c - w.shape[1])))


def _pad1(b, c):
  return jnp.pad(b, (0, c - b.shape[0])).reshape(1, c)


def _sc_mesh():
  return plsc.VectorSubcoreMesh(
      core_axis_name="c", subcore_axis_name="s", num_cores=NC,
      num_subcores=NS)


NBUF = 2


def _sc_gather(table, idx2, nout):
  rows = nout * (PE // C)
  rpw = rows // NW
  stripe = NP // NS

  def body(tab_h, idx_h, *rest):
    outs = rest[:nout]
    tab_sh, idx_v, rows_v, gsem, wsem = rest[nout:]
    c = lax.axis_index("c")
    s = lax.axis_index("s")
    wid = c * NS + s
    pltpu.sync_copy(tab_h.at[pl.ds(s * stripe, stripe)],
                    tab_sh.at[pl.ds(s * stripe, stripe)])
    pltpu.sync_copy(idx_h.at[pl.ds(wid * rpw, rpw)], idx_v)
    plsc.subcore_barrier()
    for b in range(NBUF):
      pltpu.async_copy(tab_sh.at[idx_v.at[b]], rows_v.at[b], gsem.at[b])

    def step(j, carry):
      slot = lax.rem(j, NBUF)
      pltpu.make_async_copy(tab_sh.at[idx_v.at[j]], rows_v.at[slot],
                            gsem.at[slot]).wait()
      if nout == 1:
        pltpu.async_copy(rows_v.at[slot],
                         outs[0].at[pl.ds((wid * rpw + j) * C, C)],
                         wsem.at[slot])
      else:
        @pl.when(c == 0)
        def _():
          pltpu.async_copy(rows_v.at[slot],
                           outs[0].at[pl.ds((s * rpw + j) * C, C)],
                           wsem.at[slot])

        @pl.when(c == 1)
        def _():
          pltpu.async_copy(rows_v.at[slot],
                           outs[1].at[pl.ds((s * rpw + j) * C, C)],
                           wsem.at[slot])

      k = j + NBUF

      @pl.when(k < rpw)
      def _():
        pltpu.make_async_copy(rows_v.at[slot], outs[0].at[pl.ds(0, C)],
                              wsem.at[slot]).wait()
        pltpu.async_copy(tab_sh.at[idx_v.at[k]], rows_v.at[slot],
                         gsem.at[slot])

      return carry

    lax.fori_loop(0, rpw, step, 0)
    for b in range(NBUF):
      pltpu.make_async_copy(rows_v.at[b], outs[0].at[pl.ds(0, C)],
                            wsem.at[b]).wait()

  f = pl.kernel(
      body,
      out_type=[jax.ShapeDtypeStruct((PE, HP), jnp.float32)] * nout,
      mesh=_sc_mesh(),
      scratch_types=[
          pltpu.VMEM_SHARED((NP, HP), jnp.float32),
          pltpu.VMEM((rpw, C), jnp.int32),
          pltpu.VMEM((NBUF, C, HP), jnp.float32),
          pltpu.SemaphoreType.DMA((NBUF,)),
          pltpu.SemaphoreType.DMA((NBUF,)),
      ],
  )
  return f(table, idx2)


def _sc_scatter_add(m, dst2, zeros):
  rows = PE // C
  rpw = rows // NW
  stripe = NP // NS

  def body(m_h, dst_h, z_h, out_h, acc_sh, idx_v, buf_v, rsem):
    c = lax.axis_index("c")
    s = lax.axis_index("s")
    wid = c * NS + s
    pltpu.sync_copy(z_h.at[pl.ds(s * stripe, stripe)],
                    acc_sh.at[pl.ds(s * stripe, stripe)])
    pltpu.sync_copy(dst_h.at[pl.ds(wid * rpw, rpw)], idx_v)
    plsc.subcore_barrier()
    pltpu.async_copy(m_h.at[pl.ds(wid * rpw * C, C)], buf_v.at[0],
                     rsem.at[0])

    def step(j, carry):
      slot = lax.rem(j, 2)
      row0 = (wid * rpw + j) * C
      pltpu.make_async_copy(m_h.at[pl.ds(row0, C)], buf_v.at[slot],
                            rsem.at[slot]).wait()
      k = j + 1

      @pl.when(k < rpw)
      def _():
        pltpu.async_copy(m_h.at[pl.ds((wid * rpw + k) * C, C)],
                         buf_v.at[1 - slot], rsem.at[1 - slot])

      pltpu.sync_copy(buf_v.at[slot], acc_sh.at[idx_v.at[j]], add=True)
      return carry

    lax.fori_loop(0, rpw, step, 0)
    plsc.subcore_barrier()
    pltpu.sync_copy(acc_sh.at[pl.ds(s * stripe, stripe)],
                    out_h.at[c, pl.ds(s * stripe, stripe)])

  f = pl.kernel(
      body,
      out_type=jax.ShapeDtypeStruct((NC, NP, HP), jnp.float32),
      mesh=_sc_mesh(),
      scratch_types=[
          pltpu.VMEM_SHARED((NP, HP), jnp.float32),
          pltpu.VMEM((rpw, C), jnp.int32),
          pltpu.VMEM((2, C, HP), jnp.float32),
          pltpu.SemaphoreType.DMA((2,)),
      ],
  )
  return f(m, dst2, zeros)


def _full(shape):
  return pl.BlockSpec(shape, lambda *i: (0,) * len(shape))


def _bdot(a, b):
  return jnp.dot(a.astype(jnp.bfloat16), b.astype(jnp.bfloat16),
                 preferred_element_type=jnp.float32)


def _node_embed_k(x_ref, w_ref, b_ref, o_ref):
  h = jnp.dot(x_ref[...], w_ref[...],
              preferred_element_type=jnp.float32) + b_ref[...]
  o_ref[:N, :] = h
  o_ref[N:, :] = jnp.zeros((NP - N, HP), jnp.float32)


def _node_embed(x, w, b):
  return pl.pallas_call(
      _node_embed_k,
      out_shape=jax.ShapeDtypeStruct((NP, HP), jnp.float32),
      in_specs=[_full((N, HP)), _full((HP, HP)), _full((1, HP))],
      out_specs=_full((NP, HP)),
  )(x, w, b)


def _edge_embed_msg_k(ea8_ref, hs_ref, w8_ref, eb_ref, lw_ref, lb_ref,
                      e_ref, m_ref):
  e8 = _bdot(ea8_ref[...], w8_ref[...])
  e = e8.reshape(TE, HP) + eb_ref[...]
  e_ref[...] = e
  m_ref[...] = jnp.maximum(
      hs_ref[...] + _bdot(e, lw_ref[...]) + lb_ref[...], 0.0)


def _edge_embed_msg(ea8, hs, w8, eb, lw, lb):
  g = PE // TE
  row = pl.BlockSpec((TE, HP), lambda i: (i, 0))
  return pl.pallas_call(
      _edge_embed_msg_k,
      grid=(g,),
      out_shape=[jax.ShapeDtypeStruct((PE, HP), jnp.float32),
                 jax.ShapeDtypeStruct((PE, HP), jnp.float32)],
      in_specs=[pl.BlockSpec((TE // 8, HP), lambda i: (i, 0)), row,
                _full((HP, 8 * HP)), _full((1, HP)), _full((HP, HP)),
                _full((1, HP))],
      out_specs=[row, row],
  )(ea8, hs, w8, eb, lw, lb)


def _f32(x):
  return x.astype(jnp.float32)


def _node_update_k(h_ref, ag_ref, w1_ref, b1_ref, w2_ref, b2_ref,
                   g_ref, bb_ref, o_ref):
  h = h_ref[:N, :]
  z = h + ag_ref[0, :N, :] + ag_ref[1, :N, :]
  z = jnp.maximum(jnp.dot(z, w1_ref[...],
                          preferred_element_type=jnp.float32) + b1_ref[...],
                  0.0)
  z = jnp.dot(z, w2_ref[...],
              preferred_element_type=jnp.float32) + b2_ref[...]
  mu = jnp.mean(z, axis=0, keepdims=True)
  zc = z - mu
  var = jnp.mean(zc * zc, axis=0, keepdims=True)
  zn = zc * lax.rsqrt(var + 1e-5) * g_ref[...] + bb_ref[...]
  hn = (h + jnp.maximum(zn, 0.0)) * 0.5
  o_ref[:N, :] = hn
  o_ref[N:, :] = jnp.zeros((NP - N, HP), jnp.float32)


def _node_update(h, aggr, w1, b1, w2, b2, g, bb):
  return pl.pallas_call(
      _node_update_k,
      out_shape=jax.ShapeDtypeStruct((NP, HP), jnp.float32),
      in_specs=[_full((NP, HP)), _full((NC, NP, HP)), _full((HP, HP)),
                _full((1, HP)), _full((HP, HP)), _full((1, HP)),
                _full((1, HP)), _full((1, HP))],
      out_specs=_full((NP, HP)),
  )(h, aggr, w1, b1, w2, b2, g, bb)


def _edge_update_msg_k(hs_ref, hd_ref, e_ref, w1a_ref, w1b_ref, w1c_ref,
                       b1_ref, w2_ref, b2_ref, lw_ref, lb_ref,
                       en_ref, m_ref):
  hs = hs_ref[...]
  e = e_ref[...]
  t = jnp.maximum(_bdot(hs, w1a_ref[...]) + _bdot(hd_ref[...], w1b_ref[...]) +
                  _bdot(e, w1c_ref[...]) + b1_ref[...], 0.0)
  en = e + (_bdot(t, w2_ref[...]) + b2_ref[...]) * 0.5
  en_ref[...] = en
  m_ref[...] = jnp.maximum(_f32(hs) + _bdot(en, lw_ref[...]) + lb_ref[...],
                           0.0)


def _edge_update_msg(hs, hd, e, w1a, w1b, w1c, b1, w2, b2, lw, lb):
  g = PE // TE
  row = pl.BlockSpec((TE, HP), lambda i: (i, 0))
  wspec = _full((HP, HP))
  bspec = _full((1, HP))
  return pl.pallas_call(
      _edge_update_msg_k,
      grid=(g,),
      out_shape=[jax.ShapeDtypeStruct((PE, HP), jnp.float32),
                 jax.ShapeDtypeStruct((PE, HP), jnp.float32)],
      in_specs=[row, row, row, wspec, wspec, wspec, bspec, wspec, bspec,
                wspec, bspec],
      out_specs=[row, row],
  )(hs, hd, e, w1a, w1b, w1c, b1, w2, b2, lw, lb)


def _final_k(hs_ref, hd_ref, e_ref, w1a_ref, w1b_ref, w1c_ref, b1_ref,
             w2_ref, b2_ref, m1a_ref, m1b_ref, m1c_ref, mb1_ref,
             mw2_ref, mb2_ref, mw3_ref, mb3_ref, o_ref):
  hs = hs_ref[...]
  hd = hd_ref[...]
  e = e_ref[...]
  t = jnp.maximum(_bdot(hs, w1a_ref[...]) + _bdot(hd, w1b_ref[...]) +
                  _bdot(e, w1c_ref[...]) + b1_ref[...], 0.0)
  e2 = e + (_bdot(t, w2_ref[...]) + b2_ref[...]) * 0.5
  o1 = jnp.maximum(_bdot(hs, m1a_ref[...]) + _bdot(hd, m1b_ref[...]) +
                   _bdot(e2, m1c_ref[...]) + mb1_ref[...], 0.0)
  o2 = jnp.maximum(_bdot(o1, mw2_ref[...]) + mb2_ref[...], 0.0)
  o3 = _bdot(o2, mw3_ref[...]) + mb3_ref[...]
  o_ref[...] = o3[:, 0].reshape(TE // HP, HP)


def _final(hs, hd, e, w1a, w1b, w1c, b1, w2, b2, m1a, m1b, m1c, mb1,
           mw2, mb2, mw3, mb3):
  g = PE // TE
  row = pl.BlockSpec((TE, HP), lambda i: (i, 0))
  wspec = _full((HP, HP))
  bspec = _full((1, HP))
  return pl.pallas_call(
      _final_k,
      grid=(g,),
      out_shape=jax.ShapeDtypeStruct((PE // HP, HP), jnp.float32),
      in_specs=[row, row, row, wspec, wspec, wspec, bspec, wspec, bspec,
                wspec, wspec, wspec, bspec, wspec, bspec, wspec, bspec],
      out_specs=pl.BlockSpec((TE // HP, HP), lambda i: (i, 0)),
  )(hs, hd, e, w1a, w1b, w1c, b1, w2, b2, m1a, m1b, m1c, mb1, mw2, mb2,
    mw3, mb3)


def kernel(x, edge_index, edge_attr, node_w, node_b, edge_w, edge_b,
           conv_w1, conv_b1, conv_w2, conv_b2, lin_w, lin_b,
           emlp_w1, emlp_b1, emlp_w2, emlp_b2, bn_g, bn_b,
           mlp_w1, mlp_b1, mlp_w2, mlp_b2, mlp_w3, mlp_b3):
  src = edge_index[0].astype(jnp.int32)
  dst = edge_index[1].astype(jnp.int32)
  src2 = jnp.pad(src, (0, PE - E)).reshape(PE // C, C)
  dst2 = jnp.pad(dst, (0, PE - E),
                 constant_values=N).reshape(PE // C, C)
  sd2 = jnp.concatenate([src2, dst2], axis=0)
  ea8 = jnp.pad(edge_attr.reshape(E // 8, 8 * DE),
                ((0, (PE - E) // 8), (0, 0)))
  zeros_np = jnp.zeros((NP, HP), jnp.float32)

  nw = _pad2(node_w, HP, HP)
  nb = _pad1(node_b, HP)
  ew = _pad2(edge_w, DE, HP)
  w8 = jnp.einsum('rq,fc->rfqc', jnp.eye(8, dtype=jnp.float32),
                  ew).reshape(HP, 8 * HP)
  eb = _pad1(edge_b, HP)
  lw = [_pad2(lin_w[i], HP, HP) for i in range(2)]
  lb = [_pad1(lin_b[i], HP) for i in range(2)]
  cw1 = [_pad2(conv_w1[i], HP, HP) for i in range(2)]
  cb1 = [_pad1(conv_b1[i], HP) for i in range(2)]
  cw2 = [_pad2(conv_w2[i], HP, HP) for i in range(2)]
  cb2 = [_pad1(conv_b2[i], HP) for i in range(2)]
  g_ = [_pad1(bn_g[i], HP) for i in range(2)]
  bb = [_pad1(bn_b[i], HP) for i in range(2)]
  e1a = [_pad2(emlp_w1[i][:H], HP, HP) for i in range(2)]
  e1b = [_pad2(emlp_w1[i][H:2 * H], HP, HP) for i in range(2)]
  e1c = [_pad2(emlp_w1[i][2 * H:], HP, HP) for i in range(2)]
  eb1 = [_pad1(emlp_b1[i], HP) for i in range(2)]
  ew2 = [_pad2(emlp_w2[i], HP, HP) for i in range(2)]
  eb2 = [_pad1(emlp_b2[i], HP) for i in range(2)]
  m1a = _pad2(mlp_w1[:H], HP, HP)
  m1b = _pad2(mlp_w1[H:2 * H], HP, HP)
  m1c = _pad2(mlp_w1[2 * H:], HP, HP)
  mb1 = _pad1(mlp_b1, HP)
  mw2 = _pad2(mlp_w2, HP, HP)
  mb2 = _pad1(mlp_b2, HP)
  mw3 = _pad2(mlp_w3, HP, HP)
  mb3 = _pad1(mlp_b3, HP)

  h = _node_embed(x, nw, nb)
  hs, = _sc_gather(h, src2, 1)
  e, m = _edge_embed_msg(ea8, hs, w8, eb, lw[0], lb[0])
  aggr = _sc_scatter_add(m, dst2, zeros_np)
  h = _node_update(h, aggr, cw1[0], cb1[0], cw2[0], cb2[0], g_[0], bb[0])

  hs, hd = _sc_gather(h, sd2, 2)
  e, m = _edge_update_msg(hs, hd, e, e1a[0], e1b[0], e1c[0], eb1[0],
                          ew2[0], eb2[0], lw[1], lb[1])
  aggr = _sc_scatter_add(m, dst2, zeros_np)
  h = _node_update(h, aggr, cw1[1], cb1[1], cw2[1], cb2[1], g_[1], bb[1])

  hs, hd = _sc_gather(h, sd2, 2)
  out = _final(hs, hd, e, e1a[1], e1b[1], e1c[1], eb1[1], ew2[1], eb2[1],
               m1a, m1b, m1c, mb1, mw2, mb2, mw3, mb3)
  return out.reshape(PE)[:E]


# speedup vs baseline: 1.4723x; 1.1842x over previous
import functools

import jax
import jax.numpy as jnp
from jax import lax
from jax.experimental import pallas as pl
from jax.experimental.pallas import tpu as pltpu
from jax.experimental.pallas import tpu_sc as plsc

N = 10000
E = 160000
H = 100
HP = 128
DE = 16
C = 128
PE = 163840
NP = 10240
NC, NS, NW = 2, 16, 32
TE = 2048


def _pad2(w, r, c):
  return jnp.pad(w, ((0, r - w.shape[0]), (0, c - w.shape[1])))


def _pad1(b, c):
  return jnp.pad(b, (0, c - b.shape[0])).reshape(1, c)


def _sc_mesh():
  return plsc.VectorSubcoreMesh(
      core_axis_name="c", subcore_axis_name="s", num_cores=NC,
      num_subcores=NS)


NBUF = 2


def _sc_gather(table, idx2, nout):
  rows = nout * (PE // C)
  rpw = rows // NW
  stripe = NP // NS

  def body(tab_h, idx_h, *rest):
    outs = rest[:nout]
    tab_sh, idx_v, rows_v, gsem, wsem = rest[nout:]
    c = lax.axis_index("c")
    s = lax.axis_index("s")
    wid = c * NS + s
    pltpu.sync_copy(tab_h.at[pl.ds(s * stripe, stripe)],
                    tab_sh.at[pl.ds(s * stripe, stripe)])
    pltpu.sync_copy(idx_h.at[pl.ds(wid * rpw, rpw)], idx_v)
    plsc.subcore_barrier()
    for b in range(NBUF):
      pltpu.async_copy(tab_sh.at[idx_v.at[b]], rows_v.at[b], gsem.at[b])

    def step(j, carry):
      slot = lax.rem(j, NBUF)
      pltpu.make_async_copy(tab_sh.at[idx_v.at[j]], rows_v.at[slot],
                            gsem.at[slot]).wait()
      if nout == 1:
        pltpu.async_copy(rows_v.at[slot],
                         outs[0].at[pl.ds((wid * rpw + j) * C, C)],
                         wsem.at[slot])
      else:
        @pl.when(c == 0)
        def _():
          pltpu.async_copy(rows_v.at[slot],
                           outs[0].at[pl.ds((s * rpw + j) * C, C)],
                           wsem.at[slot])

        @pl.when(c == 1)
        def _():
          pltpu.async_copy(rows_v.at[slot],
                           outs[1].at[pl.ds((s * rpw + j) * C, C)],
                           wsem.at[slot])

      k = j + NBUF

      @pl.when(k < rpw)
      def _():
        pltpu.make_async_copy(rows_v.at[slot], outs[0].at[pl.ds(0, C)],
                              wsem.at[slot]).wait()
        pltpu.async_copy(tab_sh.at[idx_v.at[k]], rows_v.at[slot],
                         gsem.at[slot])

      return carry

    lax.fori_loop(0, rpw, step, 0)
    for b in range(NBUF):
      pltpu.make_async_copy(rows_v.at[b], outs[0].at[pl.ds(0, C)],
                            wsem.at[b]).wait()

  f = pl.kernel(
      body,
      out_type=[jax.ShapeDtypeStruct((PE, HP), jnp.float32)] * nout,
      mesh=_sc_mesh(),
      scratch_types=[
          pltpu.VMEM_SHARED((NP, HP), jnp.float32),
          pltpu.VMEM((rpw, C), jnp.int32),
          pltpu.VMEM((NBUF, C, HP), jnp.float32),
          pltpu.SemaphoreType.DMA((NBUF,)),
          pltpu.SemaphoreType.DMA((NBUF,)),
      ],
  )
  return f(table, idx2)


def _sc_scatter_add(m, dst2, zeros):
  rows = PE // C
  rpw = rows // NW
  stripe = NP // NS

  def body(m_h, dst_h, z_h, out_h, acc_sh, idx_v, buf_v, rsem):
    c = lax.axis_index("c")
    s = lax.axis_index("s")
    wid = c * NS + s
    pltpu.sync_copy(z_h.at[pl.ds(s * stripe, stripe)],
                    acc_sh.at[pl.ds(s * stripe, stripe)])
    pltpu.sync_copy(dst_h.at[pl.ds(wid * rpw, rpw)], idx_v)
    plsc.subcore_barrier()
    pltpu.async_copy(m_h.at[pl.ds(wid * rpw * C, C)], buf_v.at[0],
                     rsem.at[0])

    def step(j, carry):
      slot = lax.rem(j, 2)
      row0 = (wid * rpw + j) * C
      pltpu.make_async_copy(m_h.at[pl.ds(row0, C)], buf_v.at[slot],
                            rsem.at[slot]).wait()
      k = j + 1

      @pl.when(k < rpw)
      def _():
        pltpu.async_copy(m_h.at[pl.ds((wid * rpw + k) * C, C)],
                         buf_v.at[1 - slot], rsem.at[1 - slot])

      pltpu.sync_copy(buf_v.at[slot], acc_sh.at[idx_v.at[j]], add=True)
      return carry

    lax.fori_loop(0, rpw, step, 0)
    plsc.subcore_barrier()
    pltpu.sync_copy(acc_sh.at[pl.ds(s * stripe, stripe)],
                    out_h.at[c, pl.ds(s * stripe, stripe)])

  f = pl.kernel(
      body,
      out_type=jax.ShapeDtypeStruct((NC, NP, HP), jnp.float32),
      mesh=_sc_mesh(),
      scratch_types=[
          pltpu.VMEM_SHARED((NP, HP), jnp.float32),
          pltpu.VMEM((rpw, C), jnp.int32),
          pltpu.VMEM((2, C, HP), jnp.float32),
          pltpu.SemaphoreType.DMA((2,)),
      ],
  )
  return f(m, dst2, zeros)


def _full(shape):
  return pl.BlockSpec(shape, lambda *i: (0,) * len(shape))


def _bdot(a, b):
  return jnp.dot(a.astype(jnp.bfloat16), b.astype(jnp.bfloat16),
                 preferred_element_type=jnp.float32)


def _node_embed_k(x_ref, w_ref, b_ref, o_ref):
  h = jnp.dot(x_ref[...], w_ref[...],
              preferred_element_type=jnp.float32) + b_ref[...]
  o_ref[:N, :] = h
  o_ref[N:, :] = jnp.zeros((NP - N, HP), jnp.float32)


def _node_embed(x, w, b):
  return pl.pallas_call(
      _node_embed_k,
      out_shape=jax.ShapeDtypeStruct((NP, HP), jnp.float32),
      in_specs=[_full((N, HP)), _full((HP, HP)), _full((1, HP))],
      out_specs=_full((NP, HP)),
  )(x, w, b)


def _edge_embed_msg_k(ea8_ref, hs_ref, w8_ref, eb_ref, lw_ref, lb_ref,
                      e_ref, m_ref):
  e8 = _bdot(ea8_ref[...], w8_ref[...])
  e = e8.reshape(TE, HP) + eb_ref[...]
  e_ref[...] = e
  m_ref[...] = jnp.maximum(
      hs_ref[...] + _bdot(e, lw_ref[...]) + lb_ref[...], 0.0)


def _edge_embed_msg(ea8, hs, w8, eb, lw, lb):
  g = PE // TE
  row = pl.BlockSpec((TE, HP), lambda i: (i, 0))
  return pl.pallas_call(
      _edge_embed_msg_k,
      grid=(g,),
      out_shape=[jax.ShapeDtypeStruct((PE, HP), jnp.float32),
                 jax.ShapeDtypeStruct((PE, HP), jnp.float32)],
      in_specs=[pl.BlockSpec((TE // 8, HP), lambda i: (i, 0)), row,
                _full((HP, 8 * HP)), _full((1, HP)), _full((HP, HP)),
                _full((1, HP))],
      out_specs=[row, row],
  )(ea8, hs, w8, eb, lw, lb)


def _f32(x):
  return x.astype(jnp.float32)


def _node_update_k(h_ref, ag_ref, w1_ref, b1_ref, w2_ref, b2_ref,
                   g_ref, bb_ref, o_ref):
  h = h_ref[:N, :]
  z = h + ag_ref[0, :N, :] + ag_ref[1, :N, :]
  z = jnp.maximum(jnp.dot(z, w1_ref[...],
                          preferred_element_type=jnp.float32) + b1_ref[...],
                  0.0)
  z = jnp.dot(z, w2_ref[...],
              preferred_element_type=jnp.float32) + b2_ref[...]
  mu = jnp.mean(z, axis=0, keepdims=True)
  zc = z - mu
  var = jnp.mean(zc * zc, axis=0, keepdims=True)
  zn = zc * lax.rsqrt(var + 1e-5) * g_ref[...] + bb_ref[...]
  hn = (h + jnp.maximum(zn, 0.0)) * 0.5
  o_ref[:N, :] = hn
  o_ref[N:, :] = jnp.zeros((NP - N, HP), jnp.float32)


def _node_update(h, aggr, w1, b1, w2, b2, g, bb):
  return pl.pallas_call(
      _node_update_k,
      out_shape=jax.ShapeDtypeStruct((NP, HP), jnp.float32),
      in_specs=[_full((NP, HP)), _full((NC, NP, HP)), _full((HP, HP)),
                _full((1, HP)), _full((HP, HP)), _full((1, HP)),
                _full((1, HP)), _full((1, HP))],
      out_specs=_full((NP, HP)),
  )(h, aggr, w1, b1, w2, b2, g, bb)


def _edge_update_msg_k(hs_ref, hd_ref, e_ref, w1a_ref, w1b_ref, w1c_ref,
                       b1_ref, w2_ref, b2_ref, lw_ref, lb_ref,
                       en_ref, m_ref):
  hs = hs_ref[...]
  e = e_ref[...]
  t = jnp.maximum(_bdot(hs, w1a_ref[...]) + _bdot(hd_ref[...], w1b_ref[...]) +
                  _bdot(e, w1c_ref[...]) + b1_ref[...], 0.0)
  en = e + (_bdot(t, w2_ref[...]) + b2_ref[...]) * 0.5
  en_ref[...] = en
  m_ref[...] = jnp.maximum(_f32(hs) + _bdot(en, lw_ref[...]) + lb_ref[...],
                           0.0)


def _edge_update_msg(hs, hd, e, w1a, w1b, w1c, b1, w2, b2, lw, lb):
  g = PE // TE
  row = pl.BlockSpec((TE, HP), lambda i: (i, 0))
  wspec = _full((HP, HP))
  bspec = _full((1, HP))
  return pl.pallas_call(
      _edge_update_msg_k,
      grid=(g,),
      out_shape=[jax.ShapeDtypeStruct((PE, HP), jnp.float32),
                 jax.ShapeDtypeStruct((PE, HP), jnp.float32)],
      in_specs=[row, row, row, wspec, wspec, wspec, bspec, wspec, bspec,
                wspec, bspec],
      out_specs=[row, row],
  )(hs, hd, e, w1a, w1b, w1c, b1, w2, b2, lw, lb)


def _final_k(hs_ref, hd_ref, e_ref, w1a_ref, w1b_ref, w1c_ref, b1_ref,
             w2_ref, b2_ref, m1a_ref, m1b_ref, m1c_ref, mb1_ref,
             mw2_ref, mb2_ref, mw3_ref, mb3_ref, o_ref):
  hs = hs_ref[...]
  hd = hd_ref[...]
  e = e_ref[...]
  t = jnp.maximum(_bdot(hs, w1a_ref[...]) + _bdot(hd, w1b_ref[...]) +
                  _bdot(e, w1c_ref[...]) + b1_ref[...], 0.0)
  e2 = e + (_bdot(t, w2_ref[...]) + b2_ref[...]) * 0.5
  o1 = jnp.maximum(_bdot(hs, m1a_ref[...]) + _bdot(hd, m1b_ref[...]) +
                   _bdot(e2, m1c_ref[...]) + mb1_ref[...], 0.0)
  o2 = jnp.maximum(_bdot(o1, mw2_ref[...]) + mb2_ref[...], 0.0)
  o3 = _bdot(o2, mw3_ref[...]) + mb3_ref[...]
  o_ref[...] = o3[:, 0].reshape(TE // HP, HP)


def _final(hs, hd, e, w1a, w1b, w1c, b1, w2, b2, m1a, m1b, m1c, mb1,
           mw2, mb2, mw3, mb3):
  g = PE // TE
  row = pl.BlockSpec((TE, HP), lambda i: (i, 0))
  wspec = _full((HP, HP))
  bspec = _full((1, HP))
  return pl.pallas_call(
      _final_k,
      grid=(g,),
      out_shape=jax.ShapeDtypeStruct((PE // HP, HP), jnp.float32),
      in_specs=[row, row, row, wspec, wspec, wspec, bspec, wspec, bspec,
                wspec, wspec, wspec, bspec, wspec, bspec, wspec, bspec],
      out_specs=pl.BlockSpec((TE // HP, HP), lambda i: (i, 0)),
  )(hs, hd, e, w1a, w1b, w1c, b1, w2, b2, m1a, m1b, m1c, mb1, mw2, mb2,
    mw3, mb3)


def kernel(x, edge_index, edge_attr, node_w, node_b, edge_w, edge_b,
           conv_w1, conv_b1, conv_w2, conv_b2, lin_w, lin_b,
           emlp_w1, emlp_b1, emlp_w2, emlp_b2, bn_g, bn_b,
           mlp_w1, mlp_b1, mlp_w2, mlp_b2, mlp_w3, mlp_b3):
  src = edge_index[0].astype(jnp.int32)
  dst = edge_index[1].astype(jnp.int32)
  src2 = jnp.pad(src, (0, PE - E)).reshape(PE // C, C)
  dst2 = jnp.pad(dst, (0, PE - E),
                 constant_values=N).reshape(PE // C, C)
  sd2 = jnp.concatenate([src2, dst2], axis=0)
  ea8 = jnp.pad(edge_attr.reshape(E // 8, 8 * DE),
                ((0, (PE - E) // 8), (0, 0)))
  zeros_np = jnp.zeros((NP, HP), jnp.float32)

  nw = _pad2(node_w, HP, HP)
  nb = _pad1(node_b, HP)
  ew = _pad2(edge_w, DE, HP)
  w8 = jnp.einsum('rq,fc->rfqc', jnp.eye(8, dtype=jnp.float32),
                  ew).reshape(HP, 8 * HP)
  eb = _pad1(edge_b, HP)
  lw = [_pad2(lin_w[i], HP, HP) for i in range(2)]
  lb = [_pad1(lin_b[i], HP) for i in range(2)]
  cw1 = [_pad2(conv_w1[i], HP, HP) for i in range(2)]
  cb1 = [_pad1(conv_b1[i], HP) for i in range(2)]
  cw2 = [_pad2(conv_w2[i], HP, HP) for i in range(2)]
  cb2 = [_pad1(conv_b2[i], HP) for i in range(2)]
  g_ = [_pad1(bn_g[i], HP) for i in range(2)]
  bb = [_pad1(bn_b[i], HP) for i in range(2)]
  e1a = [_pad2(emlp_w1[i][:H], HP, HP) for i in range(2)]
  e1b = [_pad2(emlp_w1[i][H:2 * H], HP, HP) for i in range(2)]
  e1c = [_pad2(emlp_w1[i][2 * H:], HP, HP) for i in range(2)]
  eb1 = [_pad1(emlp_b1[i], HP) for i in range(2)]
  ew2 = [_pad2(emlp_w2[i], HP, HP) for i in range(2)]
  eb2 = [_pad1(emlp_b2[i], HP) for i in range(2)]
  m1a = _pad2(mlp_w1[:H], HP, HP)
  m1b = _pad2(mlp_w1[H:2 * H], HP, HP)
  m1c = _pad2(mlp_w1[2 * H:], HP, HP)
  mb1 = _pad1(mlp_b1, HP)
  mw2 = _pad2(mlp_w2, HP, HP)
  mb2 = _pad1(mlp_b2, HP)
  mw3 = _pad2(mlp_w3, HP, HP)
  mb3 = _pad1(mlp_b3, HP)

  h = _node_embed(x, nw, nb)
  hs, = _sc_gather(h, src2, 1)
  e, m = _edge_embed_msg(ea8, hs, w8, eb, lw[0], lb[0])
  aggr = _sc_scatter_add(m, dst2, zeros_np)
  h = _node_update(h, aggr, cw1[0], cb1[0], cw2[0], cb2[0], g_[0], bb[0])

  hs, hd = _sc_gather(h, sd2, 2)
  e, m = _edge_update_msg(hs, hd, e, e1a[0], e1b[0], e1c[0], eb1[0],
                          ew2[0], eb2[0], lw[1], lb[1])
  aggr = _sc_scatter_add(m, dst2, zeros_np)
  h = _node_update(h, aggr, cw1[1], cb1[1], cw2[1], cb2[1], g_[1], bb[1])

  hs, hd = _sc_gather(h, sd2, 2)
  out = _final(hs, hd, e, e1a[1], e1b[1], e1c[1], eb1[1], ew2[1], eb2[1],
               m1a, m1b, m1c, mb1, mw2, mb2, mw3, mb3)
  return out.reshape(PE)[:E]


# speedup vs baseline: 1.5942x; 1.0828x over previous
import functools

import jax
import jax.numpy as jnp
from jax import lax
from jax.experimental import pallas as pl
from jax.experimental.pallas import tpu as pltpu
from jax.experimental.pallas import tpu_sc as plsc

N = 10000
E = 160000
H = 100
HP = 128
DE = 16
C = 128
PE = 163840
NP = 10240
NC, NS, NW = 2, 16, 32
TE = 4096


def _pad2(w, r, c):
  return jnp.pad(w, ((0, r - w.shape[0]), (0, c - w.shape[1])))


def _pad1(b, c):
  return jnp.pad(b, (0, c - b.shape[0])).reshape(1, c)


def _sc_mesh():
  return plsc.VectorSubcoreMesh(
      core_axis_name="c", subcore_axis_name="s", num_cores=NC,
      num_subcores=NS)


NBUF = 2


def _sc_gather(table, idx2, nout):
  rows = nout * (PE // C)
  rpw = rows // NW
  stripe = NP // NS

  def body(tab_h, idx_h, *rest):
    outs = rest[:nout]
    tab_sh, idx_v, rows_v, gsem, wsem = rest[nout:]
    c = lax.axis_index("c")
    s = lax.axis_index("s")
    wid = c * NS + s
    pltpu.sync_copy(tab_h.at[pl.ds(s * stripe, stripe)],
                    tab_sh.at[pl.ds(s * stripe, stripe)])
    pltpu.sync_copy(idx_h.at[pl.ds(wid * rpw, rpw)], idx_v)
    plsc.subcore_barrier()
    for b in range(NBUF):
      pltpu.async_copy(tab_sh.at[idx_v.at[b]], rows_v.at[b], gsem.at[b])

    def step(j, carry):
      slot = lax.rem(j, NBUF)
      pltpu.make_async_copy(tab_sh.at[idx_v.at[j]], rows_v.at[slot],
                            gsem.at[slot]).wait()
      if nout == 1:
        pltpu.async_copy(rows_v.at[slot],
                         outs[0].at[pl.ds((wid * rpw + j) * C, C)],
                         wsem.at[slot])
      else:
        @pl.when(c == 0)
        def _():
          pltpu.async_copy(rows_v.at[slot],
                           outs[0].at[pl.ds((s * rpw + j) * C, C)],
                           wsem.at[slot])

        @pl.when(c == 1)
        def _():
          pltpu.async_copy(rows_v.at[slot],
                           outs[1].at[pl.ds((s * rpw + j) * C, C)],
                           wsem.at[slot])

      k = j + NBUF

      @pl.when(k < rpw)
      def _():
        pltpu.make_async_copy(rows_v.at[slot], outs[0].at[pl.ds(0, C)],
                              wsem.at[slot]).wait()
        pltpu.async_copy(tab_sh.at[idx_v.at[k]], rows_v.at[slot],
                         gsem.at[slot])

      return carry

    lax.fori_loop(0, rpw, step, 0)
    for b in range(NBUF):
      pltpu.make_async_copy(rows_v.at[b], outs[0].at[pl.ds(0, C)],
                            wsem.at[b]).wait()

  f = pl.kernel(
      body,
      out_type=[jax.ShapeDtypeStruct((PE, HP), jnp.float32)] * nout,
      mesh=_sc_mesh(),
      scratch_types=[
          pltpu.VMEM_SHARED((NP, HP), jnp.float32),
          pltpu.VMEM((rpw, C), jnp.int32),
          pltpu.VMEM((NBUF, C, HP), jnp.float32),
          pltpu.SemaphoreType.DMA((NBUF,)),
          pltpu.SemaphoreType.DMA((NBUF,)),
      ],
  )
  return f(table, idx2)


def _sc_scatter_add(m, dst2, zeros):
  rows = PE // C
  rpw = rows // NW
  stripe = NP // NS

  def body(m_h, dst_h, z_h, out_h, acc_sh, idx_v, buf_v, rsem):
    c = lax.axis_index("c")
    s = lax.axis_index("s")
    wid = c * NS + s
    pltpu.sync_copy(z_h.at[pl.ds(s * stripe, stripe)],
                    acc_sh.at[pl.ds(s * stripe, stripe)])
    pltpu.sync_copy(dst_h.at[pl.ds(wid * rpw, rpw)], idx_v)
    plsc.subcore_barrier()
    pltpu.async_copy(m_h.at[pl.ds(wid * rpw * C, C)], buf_v.at[0],
                     rsem.at[0])

    def step(j, carry):
      slot = lax.rem(j, 2)
      row0 = (wid * rpw + j) * C
      pltpu.make_async_copy(m_h.at[pl.ds(row0, C)], buf_v.at[slot],
                            rsem.at[slot]).wait()
      k = j + 1

      @pl.when(k < rpw)
      def _():
        pltpu.async_copy(m_h.at[pl.ds((wid * rpw + k) * C, C)],
                         buf_v.at[1 - slot], rsem.at[1 - slot])

      pltpu.sync_copy(buf_v.at[slot], acc_sh.at[idx_v.at[j]], add=True)
      return carry

    lax.fori_loop(0, rpw, step, 0)
    plsc.subcore_barrier()
    pltpu.sync_copy(acc_sh.at[pl.ds(s * stripe, stripe)],
                    out_h.at[c, pl.ds(s * stripe, stripe)])

  f = pl.kernel(
      body,
      out_type=jax.ShapeDtypeStruct((NC, NP, HP), jnp.float32),
      mesh=_sc_mesh(),
      scratch_types=[
          pltpu.VMEM_SHARED((NP, HP), jnp.float32),
          pltpu.VMEM((rpw, C), jnp.int32),
          pltpu.VMEM((2, C, HP), jnp.float32),
          pltpu.SemaphoreType.DMA((2,)),
      ],
  )
  return f(m, dst2, zeros)


def _full(shape):
  return pl.BlockSpec(shape, lambda *i: (0,) * len(shape))


def _bdot(a, b):
  return jnp.dot(a.astype(jnp.bfloat16), b.astype(jnp.bfloat16),
                 preferred_element_type=jnp.float32)


def _node_embed_k(x_ref, w_ref, b_ref, o_ref):
  h = jnp.dot(x_ref[...], w_ref[...],
              preferred_element_type=jnp.float32) + b_ref[...]
  o_ref[:N, :] = h
  o_ref[N:, :] = jnp.zeros((NP - N, HP), jnp.float32)


def _node_embed(x, w, b):
  return pl.pallas_call(
      _node_embed_k,
      out_shape=jax.ShapeDtypeStruct((NP, HP), jnp.float32),
      in_specs=[_full((N, HP)), _full((HP, HP)), _full((1, HP))],
      out_specs=_full((NP, HP)),
  )(x, w, b)


def _edge_embed_msg_k(ea8_ref, hs_ref, w8_ref, eb_ref, lw_ref, lb_ref,
                      e_ref, m_ref):
  e8 = _bdot(ea8_ref[...], w8_ref[...])
  e = e8.reshape(TE, HP) + eb_ref[...]
  e_ref[...] = e
  m_ref[...] = jnp.maximum(
      hs_ref[...] + _bdot(e, lw_ref[...]) + lb_ref[...], 0.0)


def _edge_embed_msg(ea8, hs, w8, eb, lw, lb):
  g = PE // TE
  row = pl.BlockSpec((TE, HP), lambda i: (i, 0))
  return pl.pallas_call(
      _edge_embed_msg_k,
      grid=(g,),
      out_shape=[jax.ShapeDtypeStruct((PE, HP), jnp.float32),
                 jax.ShapeDtypeStruct((PE, HP), jnp.float32)],
      in_specs=[pl.BlockSpec((TE // 8, HP), lambda i: (i, 0)), row,
                _full((HP, 8 * HP)), _full((1, HP)), _full((HP, HP)),
                _full((1, HP))],
      out_specs=[row, row],
  )(ea8, hs, w8, eb, lw, lb)


def _f32(x):
  return x.astype(jnp.float32)


def _node_update_k(h_ref, ag_ref, w1_ref, b1_ref, w2_ref, b2_ref,
                   g_ref, bb_ref, o_ref):
  h = h_ref[:N, :]
  z = h + ag_ref[0, :N, :] + ag_ref[1, :N, :]
  z = jnp.maximum(jnp.dot(z, w1_ref[...],
                          preferred_element_type=jnp.float32) + b1_ref[...],
                  0.0)
  z = jnp.dot(z, w2_ref[...],
              preferred_element_type=jnp.float32) + b2_ref[...]
  mu = jnp.mean(z, axis=0, keepdims=True)
  zc = z - mu
  var = jnp.mean(zc * zc, axis=0, keepdims=True)
  zn = zc * lax.rsqrt(var + 1e-5) * g_ref[...] + bb_ref[...]
  hn = (h + jnp.maximum(zn, 0.0)) * 0.5
  o_ref[:N, :] = hn
  o_ref[N:, :] = jnp.zeros((NP - N, HP), jnp.float32)


def _node_update(h, aggr, w1, b1, w2, b2, g, bb):
  return pl.pallas_call(
      _node_update_k,
      out_shape=jax.ShapeDtypeStruct((NP, HP), jnp.float32),
      in_specs=[_full((NP, HP)), _full((NC, NP, HP)), _full((HP, HP)),
                _full((1, HP)), _full((HP, HP)), _full((1, HP)),
                _full((1, HP)), _full((1, HP))],
      out_specs=_full((NP, HP)),
  )(h, aggr, w1, b1, w2, b2, g, bb)


def _edge_update_msg_k(hs_ref, hd_ref, e_ref, w1a_ref, w1b_ref, w1c_ref,
                       b1_ref, w2_ref, b2_ref, lw_ref, lb_ref,
                       en_ref, m_ref):
  hs = hs_ref[...]
  e = e_ref[...]
  t = jnp.maximum(_bdot(hs, w1a_ref[...]) + _bdot(hd_ref[...], w1b_ref[...]) +
                  _bdot(e, w1c_ref[...]) + b1_ref[...], 0.0)
  en = e + (_bdot(t, w2_ref[...]) + b2_ref[...]) * 0.5
  en_ref[...] = en
  m_ref[...] = jnp.maximum(_f32(hs) + _bdot(en, lw_ref[...]) + lb_ref[...],
                           0.0)


def _edge_update_msg(hs, hd, e, w1a, w1b, w1c, b1, w2, b2, lw, lb):
  g = PE // TE
  row = pl.BlockSpec((TE, HP), lambda i: (i, 0))
  wspec = _full((HP, HP))
  bspec = _full((1, HP))
  return pl.pallas_call(
      _edge_update_msg_k,
      grid=(g,),
      out_shape=[jax.ShapeDtypeStruct((PE, HP), jnp.float32),
                 jax.ShapeDtypeStruct((PE, HP), jnp.float32)],
      in_specs=[row, row, row, wspec, wspec, wspec, bspec, wspec, bspec,
                wspec, bspec],
      out_specs=[row, row],
  )(hs, hd, e, w1a, w1b, w1c, b1, w2, b2, lw, lb)


def _final_k(hs_ref, hd_ref, e_ref, w1a_ref, w1b_ref, w1c_ref, b1_ref,
             w2_ref, b2_ref, m1a_ref, m1b_ref, m1c_ref, mb1_ref,
             mw2_ref, mb2_ref, mw3_ref, mb3_ref, o_ref):
  hs = hs_ref[...]
  hd = hd_ref[...]
  e = e_ref[...]
  t = jnp.maximum(_bdot(hs, w1a_ref[...]) + _bdot(hd, w1b_ref[...]) +
                  _bdot(e, w1c_ref[...]) + b1_ref[...], 0.0)
  e2 = e + (_bdot(t, w2_ref[...]) + b2_ref[...]) * 0.5
  o1 = jnp.maximum(_bdot(hs, m1a_ref[...]) + _bdot(hd, m1b_ref[...]) +
                   _bdot(e2, m1c_ref[...]) + mb1_ref[...], 0.0)
  o2 = jnp.maximum(_bdot(o1, mw2_ref[...]) + mb2_ref[...], 0.0)
  o3 = _bdot(o2, mw3_ref[...]) + mb3_ref[...]
  o_ref[...] = o3[:, 0].reshape(TE // HP, HP)


def _final(hs, hd, e, w1a, w1b, w1c, b1, w2, b2, m1a, m1b, m1c, mb1,
           mw2, mb2, mw3, mb3):
  g = PE // TE
  row = pl.BlockSpec((TE, HP), lambda i: (i, 0))
  wspec = _full((HP, HP))
  bspec = _full((1, HP))
  return pl.pallas_call(
      _final_k,
      grid=(g,),
      out_shape=jax.ShapeDtypeStruct((PE // HP, HP), jnp.float32),
      in_specs=[row, row, row, wspec, wspec, wspec, bspec, wspec, bspec,
                wspec, wspec, wspec, bspec, wspec, bspec, wspec, bspec],
      out_specs=pl.BlockSpec((TE // HP, HP), lambda i: (i, 0)),
  )(hs, hd, e, w1a, w1b, w1c, b1, w2, b2, m1a, m1b, m1c, mb1, mw2, mb2,
    mw3, mb3)


def kernel(x, edge_index, edge_attr, node_w, node_b, edge_w, edge_b,
           conv_w1, conv_b1, conv_w2, conv_b2, lin_w, lin_b,
           emlp_w1, emlp_b1, emlp_w2, emlp_b2, bn_g, bn_b,
           mlp_w1, mlp_b1, mlp_w2, mlp_b2, mlp_w3, mlp_b3):
  src = edge_index[0].astype(jnp.int32)
  dst = edge_index[1].astype(jnp.int32)
  src2 = jnp.pad(src, (0, PE - E)).reshape(PE // C, C)
  dst2 = jnp.pad(dst, (0, PE - E),
                 constant_values=N).reshape(PE // C, C)
  sd2 = jnp.concatenate([src2, dst2], axis=0)
  ea8 = jnp.pad(edge_attr.reshape(E // 8, 8 * DE),
                ((0, (PE - E) // 8), (0, 0)))
  zeros_np = jnp.zeros((NP, HP), jnp.float32)

  nw = _pad2(node_w, HP, HP)
  nb = _pad1(node_b, HP)
  ew = _pad2(edge_w, DE, HP)
  w8 = jnp.einsum('rq,fc->rfqc', jnp.eye(8, dtype=jnp.float32),
                  ew).reshape(HP, 8 * HP)
  eb = _pad1(edge_b, HP)
  lw = [_pad2(lin_w[i], HP, HP) for i in range(2)]
  lb = [_pad1(lin_b[i], HP) for i in range(2)]
  cw1 = [_pad2(conv_w1[i], HP, HP) for i in range(2)]
  cb1 = [_pad1(conv_b1[i], HP) for i in range(2)]
  cw2 = [_pad2(conv_w2[i], HP, HP) for i in range(2)]
  cb2 = [_pad1(conv_b2[i], HP) for i in range(2)]
  g_ = [_pad1(bn_g[i], HP) for i in range(2)]
  bb = [_pad1(bn_b[i], HP) for i in range(2)]
  e1a = [_pad2(emlp_w1[i][:H], HP, HP) for i in range(2)]
  e1b = [_pad2(emlp_w1[i][H:2 * H], HP, HP) for i in range(2)]
  e1c = [_pad2(emlp_w1[i][2 * H:], HP, HP) for i in range(2)]
  eb1 = [_pad1(emlp_b1[i], HP) for i in range(2)]
  ew2 = [_pad2(emlp_w2[i], HP, HP) for i in range(2)]
  eb2 = [_pad1(emlp_b2[i], HP) for i in range(2)]
  m1a = _pad2(mlp_w1[:H], HP, HP)
  m1b = _pad2(mlp_w1[H:2 * H], HP, HP)
  m1c = _pad2(mlp_w1[2 * H:], HP, HP)
  mb1 = _pad1(mlp_b1, HP)
  mw2 = _pad2(mlp_w2, HP, HP)
  mb2 = _pad1(mlp_b2, HP)
  mw3 = _pad2(mlp_w3, HP, HP)
  mb3 = _pad1(mlp_b3, HP)

  h = _node_embed(x, nw, nb)
  hs, = _sc_gather(h, src2, 1)
  e, m = _edge_embed_msg(ea8, hs, w8, eb, lw[0], lb[0])
  aggr = _sc_scatter_add(m, dst2, zeros_np)
  h = _node_update(h, aggr, cw1[0], cb1[0], cw2[0], cb2[0], g_[0], bb[0])

  hs, hd = _sc_gather(h, sd2, 2)
  e, m = _edge_update_msg(hs, hd, e, e1a[0], e1b[0], e1c[0], eb1[0],
                          ew2[0], eb2[0], lw[1], lb[1])
  aggr = _sc_scatter_add(m, dst2, zeros_np)
  h = _node_update(h, aggr, cw1[1], cb1[1], cw2[1], cb2[1], g_[1], bb[1])

  hs, hd = _sc_gather(h, sd2, 2)
  out = _final(hs, hd, e, e1a[1], e1b[1], e1c[1], eb1[1], ew2[1], eb2[1],
               m1a, m1b, m1c, mb1, mw2, mb2, mw3, mb3)
  return out.reshape(PE)[:E]


# speedup vs baseline: 1.6240x; 1.0187x over previous
import functools

import jax
import jax.numpy as jnp
from jax import lax
from jax.experimental import pallas as pl
from jax.experimental.pallas import tpu as pltpu
from jax.experimental.pallas import tpu_sc as plsc

N = 10000
E = 160000
H = 100
HP = 128
DE = 16
C = 128
PE = 163840
NP = 10240
NC, NS, NW = 2, 16, 32
TE = 8192


def _pad2(w, r, c):
  return jnp.pad(w, ((0, r - w.shape[0]), (0, c - w.shape[1])))


def _pad1(b, c):
  return jnp.pad(b, (0, c - b.shape[0])).reshape(1, c)


def _sc_mesh():
  return plsc.VectorSubcoreMesh(
      core_axis_name="c", subcore_axis_name="s", num_cores=NC,
      num_subcores=NS)


NBUF = 2


def _sc_gather(table, idx2, nout):
  rows = nout * (PE // C)
  rpw = rows // NW
  stripe = NP // NS

  def body(tab_h, idx_h, *rest):
    outs = rest[:nout]
    tab_sh, idx_v, rows_v, gsem, wsem = rest[nout:]
    c = lax.axis_index("c")
    s = lax.axis_index("s")
    wid = c * NS + s
    pltpu.sync_copy(tab_h.at[pl.ds(s * stripe, stripe)],
                    tab_sh.at[pl.ds(s * stripe, stripe)])
    pltpu.sync_copy(idx_h.at[pl.ds(wid * rpw, rpw)], idx_v)
    plsc.subcore_barrier()
    for b in range(NBUF):
      pltpu.async_copy(tab_sh.at[idx_v.at[b]], rows_v.at[b], gsem.at[b])

    def step(j, carry):
      slot = lax.rem(j, NBUF)
      pltpu.make_async_copy(tab_sh.at[idx_v.at[j]], rows_v.at[slot],
                            gsem.at[slot]).wait()
      if nout == 1:
        pltpu.async_copy(rows_v.at[slot],
                         outs[0].at[pl.ds((wid * rpw + j) * C, C)],
                         wsem.at[slot])
      else:
        @pl.when(c == 0)
        def _():
          pltpu.async_copy(rows_v.at[slot],
                           outs[0].at[pl.ds((s * rpw + j) * C, C)],
                           wsem.at[slot])

        @pl.when(c == 1)
        def _():
          pltpu.async_copy(rows_v.at[slot],
                           outs[1].at[pl.ds((s * rpw + j) * C, C)],
                           wsem.at[slot])

      k = j + NBUF

      @pl.when(k < rpw)
      def _():
        pltpu.make_async_copy(rows_v.at[slot], outs[0].at[pl.ds(0, C)],
                              wsem.at[slot]).wait()
        pltpu.async_copy(tab_sh.at[idx_v.at[k]], rows_v.at[slot],
                         gsem.at[slot])

      return carry

    lax.fori_loop(0, rpw, step, 0)
    for b in range(NBUF):
      pltpu.make_async_copy(rows_v.at[b], outs[0].at[pl.ds(0, C)],
                            wsem.at[b]).wait()

  f = pl.kernel(
      body,
      out_type=[jax.ShapeDtypeStruct((PE, HP), jnp.float32)] * nout,
      mesh=_sc_mesh(),
      scratch_types=[
          pltpu.VMEM_SHARED((NP, HP), jnp.float32),
          pltpu.VMEM((rpw, C), jnp.int32),
          pltpu.VMEM((NBUF, C, HP), jnp.float32),
          pltpu.SemaphoreType.DMA((NBUF,)),
          pltpu.SemaphoreType.DMA((NBUF,)),
      ],
  )
  return f(table, idx2)


def _sc_scatter_add(m, dst2, zeros):
  rows = PE // C
  rpw = rows // NW
  stripe = NP // NS

  def body(m_h, dst_h, z_h, out_h, acc_sh, idx_v, buf_v, rsem):
    c = lax.axis_index("c")
    s = lax.axis_index("s")
    wid = c * NS + s
    pltpu.sync_copy(z_h.at[pl.ds(s * stripe, stripe)],
                    acc_sh.at[pl.ds(s * stripe, stripe)])
    pltpu.sync_copy(dst_h.at[pl.ds(wid * rpw, rpw)], idx_v)
    plsc.subcore_barrier()
    pltpu.async_copy(m_h.at[pl.ds(wid * rpw * C, C)], buf_v.at[0],
                     rsem.at[0])

    def step(j, carry):
      slot = lax.rem(j, 2)
      row0 = (wid * rpw + j) * C
      pltpu.make_async_copy(m_h.at[pl.ds(row0, C)], buf_v.at[slot],
                            rsem.at[slot]).wait()
      k = j + 1

      @pl.when(k < rpw)
      def _():
        pltpu.async_copy(m_h.at[pl.ds((wid * rpw + k) * C, C)],
                         buf_v.at[1 - slot], rsem.at[1 - slot])

      pltpu.sync_copy(buf_v.at[slot], acc_sh.at[idx_v.at[j]], add=True)
      return carry

    lax.fori_loop(0, rpw, step, 0)
    plsc.subcore_barrier()
    pltpu.sync_copy(acc_sh.at[pl.ds(s * stripe, stripe)],
                    out_h.at[c, pl.ds(s * stripe, stripe)])

  f = pl.kernel(
      body,
      out_type=jax.ShapeDtypeStruct((NC, NP, HP), jnp.float32),
      mesh=_sc_mesh(),
      scratch_types=[
          pltpu.VMEM_SHARED((NP, HP), jnp.float32),
          pltpu.VMEM((rpw, C), jnp.int32),
          pltpu.VMEM((2, C, HP), jnp.float32),
          pltpu.SemaphoreType.DMA((2,)),
      ],
  )
  return f(m, dst2, zeros)


def _full(shape):
  return pl.BlockSpec(shape, lambda *i: (0,) * len(shape))


def _bdot(a, b):
  return jnp.dot(a.astype(jnp.bfloat16), b.astype(jnp.bfloat16),
                 preferred_element_type=jnp.float32)


def _node_embed_k(x_ref, w_ref, b_ref, o_ref):
  h = jnp.dot(x_ref[...], w_ref[...],
              preferred_element_type=jnp.float32) + b_ref[...]
  o_ref[:N, :] = h
  o_ref[N:, :] = jnp.zeros((NP - N, HP), jnp.float32)


def _node_embed(x, w, b):
  return pl.pallas_call(
      _node_embed_k,
      out_shape=jax.ShapeDtypeStruct((NP, HP), jnp.float32),
      in_specs=[_full((N, HP)), _full((HP, HP)), _full((1, HP))],
      out_specs=_full((NP, HP)),
  )(x, w, b)


def _edge_embed_msg_k(ea8_ref, hs_ref, w8_ref, eb_ref, lw_ref, lb_ref,
                      e_ref, m_ref):
  e8 = _bdot(ea8_ref[...], w8_ref[...])
  e = e8.reshape(TE, HP) + eb_ref[...]
  e_ref[...] = e
  m_ref[...] = jnp.maximum(
      hs_ref[...] + _bdot(e, lw_ref[...]) + lb_ref[...], 0.0)


def _edge_embed_msg(ea8, hs, w8, eb, lw, lb):
  g = PE // TE
  row = pl.BlockSpec((TE, HP), lambda i: (i, 0))
  return pl.pallas_call(
      _edge_embed_msg_k,
      grid=(g,),
      out_shape=[jax.ShapeDtypeStruct((PE, HP), jnp.float32),
                 jax.ShapeDtypeStruct((PE, HP), jnp.float32)],
      in_specs=[pl.BlockSpec((TE // 8, HP), lambda i: (i, 0)), row,
                _full((HP, 8 * HP)), _full((1, HP)), _full((HP, HP)),
                _full((1, HP))],
      out_specs=[row, row],
  )(ea8, hs, w8, eb, lw, lb)


def _f32(x):
  return x.astype(jnp.float32)


def _node_update_k(h_ref, ag_ref, w1_ref, b1_ref, w2_ref, b2_ref,
                   g_ref, bb_ref, o_ref):
  h = h_ref[:N, :]
  z = h + ag_ref[0, :N, :] + ag_ref[1, :N, :]
  z = jnp.maximum(jnp.dot(z, w1_ref[...],
                          preferred_element_type=jnp.float32) + b1_ref[...],
                  0.0)
  z = jnp.dot(z, w2_ref[...],
              preferred_element_type=jnp.float32) + b2_ref[...]
  mu = jnp.mean(z, axis=0, keepdims=True)
  zc = z - mu
  var = jnp.mean(zc * zc, axis=0, keepdims=True)
  zn = zc * lax.rsqrt(var + 1e-5) * g_ref[...] + bb_ref[...]
  hn = (h + jnp.maximum(zn, 0.0)) * 0.5
  o_ref[:N, :] = hn
  o_ref[N:, :] = jnp.zeros((NP - N, HP), jnp.float32)


def _node_update(h, aggr, w1, b1, w2, b2, g, bb):
  return pl.pallas_call(
      _node_update_k,
      out_shape=jax.ShapeDtypeStruct((NP, HP), jnp.float32),
      in_specs=[_full((NP, HP)), _full((NC, NP, HP)), _full((HP, HP)),
                _full((1, HP)), _full((HP, HP)), _full((1, HP)),
                _full((1, HP)), _full((1, HP))],
      out_specs=_full((NP, HP)),
  )(h, aggr, w1, b1, w2, b2, g, bb)


def _edge_update_msg_k(hs_ref, hd_ref, e_ref, w1a_ref, w1b_ref, w1c_ref,
                       b1_ref, w2_ref, b2_ref, lw_ref, lb_ref,
                       en_ref, m_ref):
  hs = hs_ref[...]
  e = e_ref[...]
  t = jnp.maximum(_bdot(hs, w1a_ref[...]) + _bdot(hd_ref[...], w1b_ref[...]) +
                  _bdot(e, w1c_ref[...]) + b1_ref[...], 0.0)
  en = e + (_bdot(t, w2_ref[...]) + b2_ref[...]) * 0.5
  en_ref[...] = en
  m_ref[...] = jnp.maximum(_f32(hs) + _bdot(en, lw_ref[...]) + lb_ref[...],
                           0.0)


def _edge_update_msg(hs, hd, e, w1a, w1b, w1c, b1, w2, b2, lw, lb):
  g = PE // TE
  row = pl.BlockSpec((TE, HP), lambda i: (i, 0))
  wspec = _full((HP, HP))
  bspec = _full((1, HP))
  return pl.pallas_call(
      _edge_update_msg_k,
      grid=(g,),
      out_shape=[jax.ShapeDtypeStruct((PE, HP), jnp.float32),
                 jax.ShapeDtypeStruct((PE, HP), jnp.float32)],
      in_specs=[row, row, row, wspec, wspec, wspec, bspec, wspec, bspec,
                wspec, bspec],
      out_specs=[row, row],
  )(hs, hd, e, w1a, w1b, w1c, b1, w2, b2, lw, lb)


def _final_k(hs_ref, hd_ref, e_ref, w1a_ref, w1b_ref, w1c_ref, b1_ref,
             w2_ref, b2_ref, m1a_ref, m1b_ref, m1c_ref, mb1_ref,
             mw2_ref, mb2_ref, mw3_ref, mb3_ref, o_ref):
  hs = hs_ref[...]
  hd = hd_ref[...]
  e = e_ref[...]
  t = jnp.maximum(_bdot(hs, w1a_ref[...]) + _bdot(hd, w1b_ref[...]) +
                  _bdot(e, w1c_ref[...]) + b1_ref[...], 0.0)
  e2 = e + (_bdot(t, w2_ref[...]) + b2_ref[...]) * 0.5
  o1 = jnp.maximum(_bdot(hs, m1a_ref[...]) + _bdot(hd, m1b_ref[...]) +
                   _bdot(e2, m1c_ref[...]) + mb1_ref[...], 0.0)
  o2 = jnp.maximum(_bdot(o1, mw2_ref[...]) + mb2_ref[...], 0.0)
  o3 = _bdot(o2, mw3_ref[...]) + mb3_ref[...]
  o_ref[...] = o3[:, 0].reshape(TE // HP, HP)


def _final(hs, hd, e, w1a, w1b, w1c, b1, w2, b2, m1a, m1b, m1c, mb1,
           mw2, mb2, mw3, mb3):
  g = PE // TE
  row = pl.BlockSpec((TE, HP), lambda i: (i, 0))
  wspec = _full((HP, HP))
  bspec = _full((1, HP))
  return pl.pallas_call(
      _final_k,
      grid=(g,),
      out_shape=jax.ShapeDtypeStruct((PE // HP, HP), jnp.float32),
      in_specs=[row, row, row, wspec, wspec, wspec, bspec, wspec, bspec,
                wspec, wspec, wspec, bspec, wspec, bspec, wspec, bspec],
      out_specs=pl.BlockSpec((TE // HP, HP), lambda i: (i, 0)),
  )(hs, hd, e, w1a, w1b, w1c, b1, w2, b2, m1a, m1b, m1c, mb1, mw2, mb2,
    mw3, mb3)


def kernel(x, edge_index, edge_attr, node_w, node_b, edge_w, edge_b,
           conv_w1, conv_b1, conv_w2, conv_b2, lin_w, lin_b,
           emlp_w1, emlp_b1, emlp_w2, emlp_b2, bn_g, bn_b,
           mlp_w1, mlp_b1, mlp_w2, mlp_b2, mlp_w3, mlp_b3):
  src = edge_index[0].astype(jnp.int32)
  dst = edge_index[1].astype(jnp.int32)
  src2 = jnp.pad(src, (0, PE - E)).reshape(PE // C, C)
  dst2 = jnp.pad(dst, (0, PE - E),
                 constant_values=N).reshape(PE // C, C)
  sd2 = jnp.concatenate([src2, dst2], axis=0)
  ea8 = jnp.pad(edge_attr.reshape(E // 8, 8 * DE),
                ((0, (PE - E) // 8), (0, 0)))
  zeros_np = jnp.zeros((NP, HP), jnp.float32)

  nw = _pad2(node_w, HP, HP)
  nb = _pad1(node_b, HP)
  ew = _pad2(edge_w, DE, HP)
  w8 = jnp.einsum('rq,fc->rfqc', jnp.eye(8, dtype=jnp.float32),
                  ew).reshape(HP, 8 * HP)
  eb = _pad1(edge_b, HP)
  lw = [_pad2(lin_w[i], HP, HP) for i in range(2)]
  lb = [_pad1(lin_b[i], HP) for i in range(2)]
  cw1 = [_pad2(conv_w1[i], HP, HP) for i in range(2)]
  cb1 = [_pad1(conv_b1[i], HP) for i in range(2)]
  cw2 = [_pad2(conv_w2[i], HP, HP) for i in range(2)]
  cb2 = [_pad1(conv_b2[i], HP) for i in range(2)]
  g_ = [_pad1(bn_g[i], HP) for i in range(2)]
  bb = [_pad1(bn_b[i], HP) for i in range(2)]
  e1a = [_pad2(emlp_w1[i][:H], HP, HP) for i in range(2)]
  e1b = [_pad2(emlp_w1[i][H:2 * H], HP, HP) for i in range(2)]
  e1c = [_pad2(emlp_w1[i][2 * H:], HP, HP) for i in range(2)]
  eb1 = [_pad1(emlp_b1[i], HP) for i in range(2)]
  ew2 = [_pad2(emlp_w2[i], HP, HP) for i in range(2)]
  eb2 = [_pad1(emlp_b2[i], HP) for i in range(2)]
  m1a = _pad2(mlp_w1[:H], HP, HP)
  m1b = _pad2(mlp_w1[H:2 * H], HP, HP)
  m1c = _pad2(mlp_w1[2 * H:], HP, HP)
  mb1 = _pad1(mlp_b1, HP)
  mw2 = _pad2(mlp_w2, HP, HP)
  mb2 = _pad1(mlp_b2, HP)
  mw3 = _pad2(mlp_w3, HP, HP)
  mb3 = _pad1(mlp_b3, HP)

  h = _node_embed(x, nw, nb)
  hs, = _sc_gather(h, src2, 1)
  e, m = _edge_embed_msg(ea8, hs, w8, eb, lw[0], lb[0])
  aggr = _sc_scatter_add(m, dst2, zeros_np)
  h = _node_update(h, aggr, cw1[0], cb1[0], cw2[0], cb2[0], g_[0], bb[0])

  hs, hd = _sc_gather(h, sd2, 2)
  e, m = _edge_update_msg(hs, hd, e, e1a[0], e1b[0], e1c[0], eb1[0],
                          ew2[0], eb2[0], lw[1], lb[1])
  aggr = _sc_scatter_add(m, dst2, zeros_np)
  h = _node_update(h, aggr, cw1[1], cb1[1], cw2[1], cb2[1], g_[1], bb[1])

  hs, hd = _sc_gather(h, sd2, 2)
  out = _final(hs, hd, e, e1a[1], e1b[1], e1c[1], eb1[1], ew2[1], eb2[1],
               m1a, m1b, m1c, mb1, mw2, mb2, mw3, mb3)
  return out.reshape(PE)[:E]
